# baseline (speedup 1.0000x reference)
import dataclasses
import functools

import jax
import jax.numpy as jnp
from jax import lax
from jax.experimental import pallas as pl
from jax.experimental.pallas import tpu as pltpu
from jax.experimental.pallas import tpu_sc as plsc

N_VERTS = 1000000
N_MAPS = 500000
PV = 1 << 20
NC = 2
NS = 16
NW = NC * NS
L = 16
ROWS = PV // 2
VB = 2048
VPS = PV // NS
PCH = VPS // VB
CHUNK = 128
CPW = 128
SLAB = CHUNK * CPW
M_PAD = SLAB * NW
NBUF = 4


def _compiler_params():
    cp = pltpu.CompilerParams()
    if "needs_layout_passes" in pltpu.CompilerParams.__dataclass_fields__:
        cp = dataclasses.replace(cp, needs_layout_passes=False)
    if "use_tc_tiling_on_sc" in pltpu.CompilerParams.__dataclass_fields__:
        cp = dataclasses.replace(cp, use_tc_tiling_on_sc=False)
    return cp


def _sc_loss_parts(planes, fidx, tidx):
    mesh = plsc.VectorSubcoreMesh(core_axis_name="c", subcore_axis_name="s")

    @functools.partial(
        pl.kernel,
        mesh=mesh,
        compiler_params=_compiler_params(),
        out_type=(
            jax.ShapeDtypeStruct((NW, L), jnp.float32),
            jax.ShapeDtypeStruct((NC, ROWS, 8), jnp.float32),
        ),
        scratch_types=[
            pltpu.VMEM((SLAB,), jnp.int32),
            pltpu.VMEM((SLAB,), jnp.int32),
            pltpu.VMEM((CHUNK,), jnp.int32),
            pltpu.VMEM((CHUNK,), jnp.int32),
            pltpu.VMEM((CHUNK,), jnp.int32),
            pltpu.VMEM((CHUNK,), jnp.int32),
            pltpu.VMEM((CHUNK,), jnp.int32),
            pltpu.VMEM((CHUNK,), jnp.int32),
            pltpu.VMEM((CHUNK,), jnp.int32),
            pltpu.VMEM((CHUNK,), jnp.int32),
            pltpu.VMEM((CHUNK, 8), jnp.float32),
            pltpu.VMEM((CHUNK, 8), jnp.float32),
            pltpu.VMEM((CHUNK, 8), jnp.float32),
            pltpu.VMEM((CHUNK, 8), jnp.float32),
            pltpu.VMEM((CHUNK, 8), jnp.float32),
            pltpu.VMEM((CHUNK, 8), jnp.float32),
            pltpu.VMEM((CHUNK, 8), jnp.float32),
            pltpu.VMEM((CHUNK, 8), jnp.float32),
            pltpu.VMEM((3, VB), jnp.float32),
            pltpu.VMEM((3, VB), jnp.float32),
            pltpu.VMEM((VB // 2, 8), jnp.float32),
            pltpu.VMEM((VB // 2, 8), jnp.float32),
            pltpu.VMEM((L,), jnp.float32),
            pltpu.SemaphoreType.DMA,
            pltpu.SemaphoreType.DMA,
            pltpu.SemaphoreType.DMA,
            pltpu.SemaphoreType.DMA,
            pltpu.SemaphoreType.DMA,
            pltpu.SemaphoreType.DMA,
            pltpu.SemaphoreType.DMA,
            pltpu.SemaphoreType.DMA,
            pltpu.SemaphoreType.DMA,
        ],
    )
    def k(pl_hbm, f_hbm, t_hbm, out_hbm, tab_hbm, fi_v, ti_v,
          fs0, fs1, fs2, fs3, ts0, ts1, ts2, ts3,
          fr_v0, fr_v1, fr_v2, fr_v3, tr_v0, tr_v1, tr_v2, tr_v3,
          pin0, pin1, po0, po1, acc_v,
          sem_i, sem_g0, sem_g1, sem_g2, sem_g3,
          sem_p0, sem_p1, sem_o0, sem_o1):
        cid = lax.axis_index("c")
        sid = lax.axis_index("s")
        wid = sid * NC + cid
        base = wid * SLAB
        gsems = (sem_g0, sem_g1, sem_g2, sem_g3)
        psems = (sem_p0, sem_p1)
        osems = (sem_o0, sem_o1)
        fss = (fs0, fs1, fs2, fs3)
        tss = (ts0, ts1, ts2, ts3)
        frs = (fr_v0, fr_v1, fr_v2, fr_v3)
        trs = (tr_v0, tr_v1, tr_v2, tr_v3)
        pins = (pin0, pin1)
        pos = (po0, po1)
        acc_v[...] = jnp.zeros((L,), jnp.float32)
        rows16 = lax.iota(jnp.int32, 16)
        mytab = tab_hbm.at[cid]

        pltpu.async_copy(f_hbm.at[pl.ds(base, SLAB)], fi_v, sem_i)
        pltpu.async_copy(t_hbm.at[pl.ds(base, SLAB)], ti_v, sem_i)

        vbase = sid * VPS

        def p1_issue_in(p, slot):
            v0 = vbase + p * VB
            for comp in range(3):
                pltpu.async_copy(
                    pl_hbm.at[pl.ds(comp * PV + v0, VB)],
                    pins[slot].at[comp], psems[slot])

        def p1_drain_in(slot):
            for comp in range(3):
                pltpu.make_async_copy(
                    pl_hbm.at[pl.ds(0, VB)], pins[slot].at[comp],
                    psems[slot]).wait()

        def p1_drain_out(slot):
            pltpu.make_async_copy(
                pos[slot], mytab.at[pl.ds(0, VB // 2)], osems[slot]).wait()

        rp = jnp.right_shift(rows16, 1)
        cp = jnp.bitwise_and(rows16, 1)
        zrow = jnp.right_shift(rows16, 1)
        zcol = jnp.bitwise_and(rows16, 1) + 6
        zeros16 = jnp.zeros((L,), jnp.float32)

        p1_issue_in(0, 0)
        p1_issue_in(1, 1)

        @pl.loop(0, PCH, step=2)
        def _(p):
            for b in range(2):
                step = p + b
                p1_drain_in(b)

                @pl.when(step >= 2)
                def _():
                    p1_drain_out(b)

                pin = pins[b]
                po = pos[b]

                @pl.loop(0, VB, step=L)
                def _(g):
                    xv = pin[0, pl.ds(g, L)]
                    yv = pin[1, pl.ds(g, L)]
                    zv = pin[2, pl.ds(g, L)]
                    rr = rp + jnp.right_shift(g, 1)
                    plsc.store_scatter(po, [rr, cp], xv)
                    plsc.store_scatter(po, [rr, cp + 2], yv)
                    plsc.store_scatter(po, [rr, cp + 4], zv)
                    plsc.store_scatter(
                        po, [zrow + jnp.right_shift(g, 1), zcol], zeros16)
                row0 = (vbase + step * VB) // 2
                pltpu.async_copy(
                    po, mytab.at[pl.ds(row0, VB // 2)], osems[b])

                @pl.when(step + 2 < PCH)
                def _():
                    p1_issue_in(step + 2, b)

        p1_drain_out(0)
        p1_drain_out(1)
        plsc.subcore_barrier()

        pltpu.make_async_copy(f_hbm.at[pl.ds(base, SLAB)], fi_v, sem_i).wait()
        pltpu.make_async_copy(t_hbm.at[pl.ds(base, SLAB)], ti_v, sem_i).wait()

        def stage_issue(chunk, slot):
            off = chunk * CHUNK

            @pl.loop(0, CHUNK, step=L)
            def _(g):
                fss[slot][pl.ds(g, L)] = jnp.right_shift(
                    fi_v[pl.ds(off + g, L)], 1)
                tss[slot][pl.ds(g, L)] = jnp.right_shift(
                    ti_v[pl.ds(off + g, L)], 1)
            pltpu.async_copy(mytab.at[fss[slot]], frs[slot], gsems[slot])
            pltpu.async_copy(mytab.at[tss[slot]], trs[slot], gsems[slot])

        def drain(slot):
            pltpu.make_async_copy(
                mytab.at[fss[slot]], frs[slot], gsems[slot]).wait()
            pltpu.make_async_copy(
                mytab.at[tss[slot]], trs[slot], gsems[slot]).wait()

        for _s in range(NBUF):
            stage_issue(_s, _s)

        @pl.loop(0, CPW, step=NBUF)
        def _(j):
            for b in range(NBUF):
                chunk = j + b
                drain(b)
                fr = frs[b]
                tr = trs[b]
                off = chunk * CHUNK

                @pl.loop(0, CHUNK, step=L)
                def _(r):
                    ridx = rows16 + r
                    mf = jnp.bitwise_and(fi_v[pl.ds(off + r, L)], 1)
                    mt = jnp.bitwise_and(ti_v[pl.ds(off + r, L)], 1)
                    xf = plsc.load_gather(fr, [ridx, mf])
                    yf = plsc.load_gather(fr, [ridx, mf + 2])
                    zf = plsc.load_gather(fr, [ridx, mf + 4])
                    xt = plsc.load_gather(tr, [ridx, mt])
                    yt = plsc.load_gather(tr, [ridx, mt + 2])
                    zt = plsc.load_gather(tr, [ridx, mt + 4])
                    sx = xt + xf
                    sy = yt - yf
                    sz = zt - zf
                    ss = sx * sx + sy * sy + sz * sz
                    ss = jnp.maximum(ss, 1e-30)
                    bits = plsc.bitcast(ss, jnp.int32)
                    bits = 0x5F3759DF - jnp.right_shift(bits, 1)
                    y = plsc.bitcast(bits, jnp.float32)
                    for _ in range(2):
                        y = y * (1.5 - 0.5 * ss * y * y)
                    acc_v[...] = acc_v[...] + ss * y

                @pl.when(chunk + NBUF < CPW)
                def _():
                    stage_issue(chunk + NBUF, b)

        pltpu.async_copy(acc_v, out_hbm.at[wid], sem_i).wait()

    return k(planes, fidx, tidx)


def kernel(vt, mapping_table):
    planes = jnp.pad(vt.T, ((0, 0), (0, PV - N_VERTS))).reshape(3 * PV)
    pad = jnp.full((M_PAD - N_MAPS,), N_VERTS, jnp.int32)
    fidx = jnp.concatenate([mapping_table[:, 0], pad])
    tidx = jnp.concatenate([mapping_table[:, 1], pad])
    parts, _ = _sc_loss_parts(planes, fidx, tidx)
    return jnp.sum(parts) / N_MAPS


# speedup vs baseline: 1.0195x; 1.0195x over previous
import dataclasses
import functools

import jax
import jax.numpy as jnp
from jax import lax
from jax.experimental import pallas as pl
from jax.experimental.pallas import tpu as pltpu
from jax.experimental.pallas import tpu_sc as plsc

N_VERTS = 1000000
N_MAPS = 500000
PV = 1 << 20
NC = 2
NS = 16
NW = NC * NS
L = 16
ROWS = PV // 2
VB = 2048
VPS = PV // NS
PCH = VPS // VB
CHUNK = 128
CPW = 128
SLAB = CHUNK * CPW
M_PAD = SLAB * NW
NBUF = 2


def _compiler_params():
    cp = pltpu.CompilerParams()
    if "needs_layout_passes" in pltpu.CompilerParams.__dataclass_fields__:
        cp = dataclasses.replace(cp, needs_layout_passes=False)
    if "use_tc_tiling_on_sc" in pltpu.CompilerParams.__dataclass_fields__:
        cp = dataclasses.replace(cp, use_tc_tiling_on_sc=False)
    return cp


def _sc_loss_parts(planes, fidx, tidx):
    mesh = plsc.VectorSubcoreMesh(core_axis_name="c", subcore_axis_name="s")

    @functools.partial(
        pl.kernel,
        mesh=mesh,
        compiler_params=_compiler_params(),
        out_type=(
            jax.ShapeDtypeStruct((NW, L), jnp.float32),
            jax.ShapeDtypeStruct((NC, ROWS, 8), jnp.float32),
        ),
        scratch_types=[
            pltpu.VMEM((SLAB,), jnp.int32),
            pltpu.VMEM((SLAB,), jnp.int32),
            pltpu.VMEM((CHUNK,), jnp.int32),
            pltpu.VMEM((CHUNK,), jnp.int32),
            pltpu.VMEM((CHUNK,), jnp.int32),
            pltpu.VMEM((CHUNK,), jnp.int32),
            pltpu.VMEM((CHUNK, 8), jnp.float32),
            pltpu.VMEM((CHUNK, 8), jnp.float32),
            pltpu.VMEM((CHUNK, 8), jnp.float32),
            pltpu.VMEM((CHUNK, 8), jnp.float32),
            pltpu.VMEM((3, VB), jnp.float32),
            pltpu.VMEM((3, VB), jnp.float32),
            pltpu.VMEM((VB // 2, 8), jnp.float32),
            pltpu.VMEM((VB // 2, 8), jnp.float32),
            pltpu.VMEM((L,), jnp.float32),
            pltpu.SemaphoreType.DMA,
            pltpu.SemaphoreType.DMA,
            pltpu.SemaphoreType.DMA,
            pltpu.SemaphoreType.DMA,
            pltpu.SemaphoreType.DMA,
            pltpu.SemaphoreType.DMA,
            pltpu.SemaphoreType.DMA,
        ],
    )
    def k(pl_hbm, f_hbm, t_hbm, out_hbm, tab_hbm, fi_v, ti_v,
          fs0, fs1, ts0, ts1, fr_v0, fr_v1, tr_v0, tr_v1,
          pin0, pin1, po0, po1, acc_v,
          sem_i, sem_g0, sem_g1, sem_p0, sem_p1, sem_o0, sem_o1):
        cid = lax.axis_index("c")
        sid = lax.axis_index("s")
        wid = sid * NC + cid
        base = wid * SLAB
        gsems = (sem_g0, sem_g1)
        psems = (sem_p0, sem_p1)
        osems = (sem_o0, sem_o1)
        fss = (fs0, fs1)
        tss = (ts0, ts1)
        frs = (fr_v0, fr_v1)
        trs = (tr_v0, tr_v1)
        pins = (pin0, pin1)
        pos = (po0, po1)
        acc_v[...] = jnp.zeros((L,), jnp.float32)
        rows16 = lax.iota(jnp.int32, 16)
        mytab = tab_hbm.at[cid]

        pltpu.async_copy(f_hbm.at[pl.ds(base, SLAB)], fi_v, sem_i)
        pltpu.async_copy(t_hbm.at[pl.ds(base, SLAB)], ti_v, sem_i)

        vbase = sid * VPS

        def p1_issue_in(p, slot):
            v0 = vbase + p * VB
            for comp in range(3):
                pltpu.async_copy(
                    pl_hbm.at[pl.ds(comp * PV + v0, VB)],
                    pins[slot].at[comp], psems[slot])

        def p1_drain_in(slot):
            for comp in range(3):
                pltpu.make_async_copy(
                    pl_hbm.at[pl.ds(0, VB)], pins[slot].at[comp],
                    psems[slot]).wait()

        def p1_drain_out(slot):
            pltpu.make_async_copy(
                pos[slot], mytab.at[pl.ds(0, VB // 2)], osems[slot]).wait()

        rp = jnp.right_shift(rows16, 1)
        cp = jnp.bitwise_and(rows16, 1)
        zrow = jnp.right_shift(rows16, 1)
        zcol = jnp.bitwise_and(rows16, 1) + 6
        zeros16 = jnp.zeros((L,), jnp.float32)

        p1_issue_in(0, 0)
        p1_issue_in(1, 1)

        @pl.loop(0, PCH, step=NBUF)
        def _(p):
            for b in range(NBUF):
                step = p + b
                p1_drain_in(b)

                @pl.when(step >= NBUF)
                def _():
                    p1_drain_out(b)

                pin = pins[b]
                po = pos[b]

                @pl.loop(0, VB, step=L)
                def _(g):
                    xv = pin[0, pl.ds(g, L)]
                    yv = pin[1, pl.ds(g, L)]
                    zv = pin[2, pl.ds(g, L)]
                    rr = rp + jnp.right_shift(g, 1)
                    plsc.store_scatter(po, [rr, cp], xv)
                    plsc.store_scatter(po, [rr, cp + 2], yv)
                    plsc.store_scatter(po, [rr, cp + 4], zv)
                    plsc.store_scatter(
                        po, [zrow + jnp.right_shift(g, 1), zcol], zeros16)
                row0 = (vbase + step * VB) // 2
                pltpu.async_copy(
                    po, mytab.at[pl.ds(row0, VB // 2)], osems[b])

                @pl.when(step + NBUF < PCH)
                def _():
                    p1_issue_in(step + NBUF, b)

        p1_drain_out(0)
        p1_drain_out(1)
        plsc.subcore_barrier()

        pltpu.make_async_copy(f_hbm.at[pl.ds(base, SLAB)], fi_v, sem_i).wait()
        pltpu.make_async_copy(t_hbm.at[pl.ds(base, SLAB)], ti_v, sem_i).wait()

        def stage_issue(chunk, slot):
            off = chunk * CHUNK

            @pl.loop(0, CHUNK, step=L)
            def _(g):
                fss[slot][pl.ds(g, L)] = jnp.right_shift(
                    fi_v[pl.ds(off + g, L)], 1)
                tss[slot][pl.ds(g, L)] = jnp.right_shift(
                    ti_v[pl.ds(off + g, L)], 1)
            pltpu.async_copy(mytab.at[fss[slot]], frs[slot], gsems[slot])
            pltpu.async_copy(mytab.at[tss[slot]], trs[slot], gsems[slot])

        def drain(slot):
            pltpu.make_async_copy(
                mytab.at[fss[slot]], frs[slot], gsems[slot]).wait()
            pltpu.make_async_copy(
                mytab.at[tss[slot]], trs[slot], gsems[slot]).wait()

        stage_issue(0, 0)
        stage_issue(1, 1)

        @pl.loop(0, CPW, step=NBUF)
        def _(j):
            for b in range(NBUF):
                chunk = j + b
                drain(b)
                fr = frs[b]
                tr = trs[b]
                off = chunk * CHUNK

                @pl.loop(0, CHUNK, step=L)
                def _(r):
                    ridx = rows16 + r
                    mf = jnp.bitwise_and(fi_v[pl.ds(off + r, L)], 1)
                    mt = jnp.bitwise_and(ti_v[pl.ds(off + r, L)], 1)
                    xf = plsc.load_gather(fr, [ridx, mf])
                    yf = plsc.load_gather(fr, [ridx, mf + 2])
                    zf = plsc.load_gather(fr, [ridx, mf + 4])
                    xt = plsc.load_gather(tr, [ridx, mt])
                    yt = plsc.load_gather(tr, [ridx, mt + 2])
                    zt = plsc.load_gather(tr, [ridx, mt + 4])
                    sx = xt + xf
                    sy = yt - yf
                    sz = zt - zf
                    ss = sx * sx + sy * sy + sz * sz
                    ss = jnp.maximum(ss, 1e-30)
                    bits = plsc.bitcast(ss, jnp.int32)
                    bits = 0x5F3759DF - jnp.right_shift(bits, 1)
                    y = plsc.bitcast(bits, jnp.float32)
                    for _ in range(2):
                        y = y * (1.5 - 0.5 * ss * y * y)
                    acc_v[...] = acc_v[...] + ss * y

                @pl.when(chunk + NBUF < CPW)
                def _():
                    stage_issue(chunk + NBUF, b)

        pltpu.async_copy(acc_v, out_hbm.at[wid], sem_i).wait()

    return k(planes, fidx, tidx)


def kernel(vt, mapping_table):
    planes = jnp.pad(vt.T, ((0, 0), (0, PV - N_VERTS))).reshape(3 * PV)
    pad = jnp.full((M_PAD - N_MAPS,), N_VERTS, jnp.int32)
    fidx = jnp.concatenate([mapping_table[:, 0], pad])
    tidx = jnp.concatenate([mapping_table[:, 1], pad])
    parts, _ = _sc_loss_parts(planes, fidx, tidx)
    return jnp.sum(parts) / N_MAPS


# speedup vs baseline: 1.0363x; 1.0165x over previous
import dataclasses
import functools

import jax
import jax.numpy as jnp
from jax import lax
from jax.experimental import pallas as pl
from jax.experimental.pallas import tpu as pltpu
from jax.experimental.pallas import tpu_sc as plsc

N_VERTS = 1000000
N_MAPS = 500000
PV = 1 << 20
NC = 2
NS = 16
NW = NC * NS
L = 16
ROWS = PV // 2
VB = 4096
VPS = PV // NS
PCH = VPS // VB
CHUNK = 128
CPW = 128
SLAB = CHUNK * CPW
M_PAD = SLAB * NW
NBUF = 2


def _compiler_params():
    cp = pltpu.CompilerParams()
    if "needs_layout_passes" in pltpu.CompilerParams.__dataclass_fields__:
        cp = dataclasses.replace(cp, needs_layout_passes=False)
    if "use_tc_tiling_on_sc" in pltpu.CompilerParams.__dataclass_fields__:
        cp = dataclasses.replace(cp, use_tc_tiling_on_sc=False)
    return cp


def _sc_loss_parts(planes, fidx, tidx):
    mesh = plsc.VectorSubcoreMesh(core_axis_name="c", subcore_axis_name="s")

    @functools.partial(
        pl.kernel,
        mesh=mesh,
        compiler_params=_compiler_params(),
        out_type=(
            jax.ShapeDtypeStruct((NW, L), jnp.float32),
            jax.ShapeDtypeStruct((NC, ROWS, 8), jnp.float32),
        ),
        scratch_types=[
            pltpu.VMEM((SLAB,), jnp.int32),
            pltpu.VMEM((SLAB,), jnp.int32),
            pltpu.VMEM((CHUNK,), jnp.int32),
            pltpu.VMEM((CHUNK,), jnp.int32),
            pltpu.VMEM((CHUNK,), jnp.int32),
            pltpu.VMEM((CHUNK,), jnp.int32),
            pltpu.VMEM((CHUNK, 8), jnp.float32),
            pltpu.VMEM((CHUNK, 8), jnp.float32),
            pltpu.VMEM((CHUNK, 8), jnp.float32),
            pltpu.VMEM((CHUNK, 8), jnp.float32),
            pltpu.VMEM((3, VB), jnp.float32),
            pltpu.VMEM((3, VB), jnp.float32),
            pltpu.VMEM((VB // 2, 8), jnp.float32),
            pltpu.VMEM((VB // 2, 8), jnp.float32),
            pltpu.VMEM((L,), jnp.float32),
            pltpu.SemaphoreType.DMA,
            pltpu.SemaphoreType.DMA,
            pltpu.SemaphoreType.DMA,
            pltpu.SemaphoreType.DMA,
            pltpu.SemaphoreType.DMA,
            pltpu.SemaphoreType.DMA,
            pltpu.SemaphoreType.DMA,
        ],
    )
    def k(pl_hbm, f_hbm, t_hbm, out_hbm, tab_hbm, fi_v, ti_v,
          fs0, fs1, ts0, ts1, fr_v0, fr_v1, tr_v0, tr_v1,
          pin0, pin1, po0, po1, acc_v,
          sem_i, sem_g0, sem_g1, sem_p0, sem_p1, sem_o0, sem_o1):
        cid = lax.axis_index("c")
        sid = lax.axis_index("s")
        wid = sid * NC + cid
        base = wid * SLAB
        gsems = (sem_g0, sem_g1)
        psems = (sem_p0, sem_p1)
        osems = (sem_o0, sem_o1)
        fss = (fs0, fs1)
        tss = (ts0, ts1)
        frs = (fr_v0, fr_v1)
        trs = (tr_v0, tr_v1)
        pins = (pin0, pin1)
        pos = (po0, po1)
        acc_v[...] = jnp.zeros((L,), jnp.float32)
        rows16 = lax.iota(jnp.int32, 16)
        mytab = tab_hbm.at[cid]

        pltpu.async_copy(f_hbm.at[pl.ds(base, SLAB)], fi_v, sem_i)
        pltpu.async_copy(t_hbm.at[pl.ds(base, SLAB)], ti_v, sem_i)

        vbase = sid * VPS

        def p1_issue_in(p, slot):
            v0 = vbase + p * VB
            for comp in range(3):
                pltpu.async_copy(
                    pl_hbm.at[pl.ds(comp * PV + v0, VB)],
                    pins[slot].at[comp], psems[slot])

        def p1_drain_in(slot):
            for comp in range(3):
                pltpu.make_async_copy(
                    pl_hbm.at[pl.ds(0, VB)], pins[slot].at[comp],
                    psems[slot]).wait()

        def p1_drain_out(slot):
            pltpu.make_async_copy(
                pos[slot], mytab.at[pl.ds(0, VB // 2)], osems[slot]).wait()

        rp = jnp.right_shift(rows16, 1)
        cp = jnp.bitwise_and(rows16, 1)
        zrow = jnp.right_shift(rows16, 1)
        zcol = jnp.bitwise_and(rows16, 1) + 6
        zeros16 = jnp.zeros((L,), jnp.float32)

        p1_issue_in(0, 0)
        p1_issue_in(1, 1)

        @pl.loop(0, PCH, step=NBUF)
        def _(p):
            for b in range(NBUF):
                step = p + b
                p1_drain_in(b)

                @pl.when(step >= NBUF)
                def _():
                    p1_drain_out(b)

                pin = pins[b]
                po = pos[b]

                @pl.loop(0, VB, step=L)
                def _(g):
                    xv = pin[0, pl.ds(g, L)]
                    yv = pin[1, pl.ds(g, L)]
                    zv = pin[2, pl.ds(g, L)]
                    rr = rp + jnp.right_shift(g, 1)
                    plsc.store_scatter(po, [rr, cp], xv)
                    plsc.store_scatter(po, [rr, cp + 2], yv)
                    plsc.store_scatter(po, [rr, cp + 4], zv)
                    plsc.store_scatter(
                        po, [zrow + jnp.right_shift(g, 1), zcol], zeros16)
                row0 = (vbase + step * VB) // 2
                pltpu.async_copy(
                    po, mytab.at[pl.ds(row0, VB // 2)], osems[b])

                @pl.when(step + NBUF < PCH)
                def _():
                    p1_issue_in(step + NBUF, b)

        p1_drain_out(0)
        p1_drain_out(1)
        plsc.subcore_barrier()

        pltpu.make_async_copy(f_hbm.at[pl.ds(base, SLAB)], fi_v, sem_i).wait()
        pltpu.make_async_copy(t_hbm.at[pl.ds(base, SLAB)], ti_v, sem_i).wait()

        def stage_issue(chunk, slot):
            off = chunk * CHUNK

            @pl.loop(0, CHUNK, step=L)
            def _(g):
                fss[slot][pl.ds(g, L)] = jnp.right_shift(
                    fi_v[pl.ds(off + g, L)], 1)
                tss[slot][pl.ds(g, L)] = jnp.right_shift(
                    ti_v[pl.ds(off + g, L)], 1)
            pltpu.async_copy(mytab.at[fss[slot]], frs[slot], gsems[slot])
            pltpu.async_copy(mytab.at[tss[slot]], trs[slot], gsems[slot])

        def drain(slot):
            pltpu.make_async_copy(
                mytab.at[fss[slot]], frs[slot], gsems[slot]).wait()
            pltpu.make_async_copy(
                mytab.at[tss[slot]], trs[slot], gsems[slot]).wait()

        stage_issue(0, 0)
        stage_issue(1, 1)

        @pl.loop(0, CPW, step=NBUF)
        def _(j):
            for b in range(NBUF):
                chunk = j + b
                drain(b)
                fr = frs[b]
                tr = trs[b]
                off = chunk * CHUNK

                @pl.loop(0, CHUNK, step=L)
                def _(r):
                    ridx = rows16 + r
                    mf = jnp.bitwise_and(fi_v[pl.ds(off + r, L)], 1)
                    mt = jnp.bitwise_and(ti_v[pl.ds(off + r, L)], 1)
                    xf = plsc.load_gather(fr, [ridx, mf])
                    yf = plsc.load_gather(fr, [ridx, mf + 2])
                    zf = plsc.load_gather(fr, [ridx, mf + 4])
                    xt = plsc.load_gather(tr, [ridx, mt])
                    yt = plsc.load_gather(tr, [ridx, mt + 2])
                    zt = plsc.load_gather(tr, [ridx, mt + 4])
                    sx = xt + xf
                    sy = yt - yf
                    sz = zt - zf
                    ss = sx * sx + sy * sy + sz * sz
                    ss = jnp.maximum(ss, 1e-30)
                    bits = plsc.bitcast(ss, jnp.int32)
                    bits = 0x5F3759DF - jnp.right_shift(bits, 1)
                    y = plsc.bitcast(bits, jnp.float32)
                    for _ in range(2):
                        y = y * (1.5 - 0.5 * ss * y * y)
                    acc_v[...] = acc_v[...] + ss * y

                @pl.when(chunk + NBUF < CPW)
                def _():
                    stage_issue(chunk + NBUF, b)

        pltpu.async_copy(acc_v, out_hbm.at[wid], sem_i).wait()

    return k(planes, fidx, tidx)


def kernel(vt, mapping_table):
    planes = jnp.pad(vt.T, ((0, 0), (0, PV - N_VERTS))).reshape(3 * PV)
    pad = jnp.full((M_PAD - N_MAPS,), N_VERTS, jnp.int32)
    fidx = jnp.concatenate([mapping_table[:, 0], pad])
    tidx = jnp.concatenate([mapping_table[:, 1], pad])
    parts, _ = _sc_loss_parts(planes, fidx, tidx)
    return jnp.sum(parts) / N_MAPS


# speedup vs baseline: 1.0394x; 1.0030x over previous
import dataclasses
import functools

import jax
import jax.numpy as jnp
from jax import lax
from jax.experimental import pallas as pl
from jax.experimental.pallas import tpu as pltpu
from jax.experimental.pallas import tpu_sc as plsc

N_VERTS = 1000000
N_MAPS = 500000
PV = 1 << 20
NC = 2
NS = 16
NW = NC * NS
L = 16
ROWS = PV // 2
VB = 4096
VPS = PV // NS
PCH = VPS // VB
CHUNK = 128
CPW = 128
SLAB = CHUNK * CPW
M_PAD = SLAB * NW
NBUF = 2


def _compiler_params():
    cp = pltpu.CompilerParams()
    if "needs_layout_passes" in pltpu.CompilerParams.__dataclass_fields__:
        cp = dataclasses.replace(cp, needs_layout_passes=False)
    if "use_tc_tiling_on_sc" in pltpu.CompilerParams.__dataclass_fields__:
        cp = dataclasses.replace(cp, use_tc_tiling_on_sc=False)
    return cp


def _sc_loss_parts(planes, fidx, tidx):
    mesh = plsc.VectorSubcoreMesh(core_axis_name="c", subcore_axis_name="s")

    @functools.partial(
        pl.kernel,
        mesh=mesh,
        compiler_params=_compiler_params(),
        out_type=(
            jax.ShapeDtypeStruct((NW, L), jnp.float32),
            jax.ShapeDtypeStruct((NC, ROWS, 8), jnp.float32),
        ),
        scratch_types=[
            pltpu.VMEM((SLAB,), jnp.int32),
            pltpu.VMEM((SLAB,), jnp.int32),
            pltpu.VMEM((CHUNK,), jnp.int32),
            pltpu.VMEM((CHUNK,), jnp.int32),
            pltpu.VMEM((CHUNK,), jnp.int32),
            pltpu.VMEM((CHUNK,), jnp.int32),
            pltpu.VMEM((CHUNK, 8), jnp.float32),
            pltpu.VMEM((CHUNK, 8), jnp.float32),
            pltpu.VMEM((CHUNK, 8), jnp.float32),
            pltpu.VMEM((CHUNK, 8), jnp.float32),
            pltpu.VMEM((3, VB), jnp.float32),
            pltpu.VMEM((3, VB), jnp.float32),
            pltpu.VMEM((VB // 2, 8), jnp.float32),
            pltpu.VMEM((VB // 2, 8), jnp.float32),
            pltpu.VMEM((L,), jnp.float32),
            pltpu.SemaphoreType.DMA,
            pltpu.SemaphoreType.DMA,
            pltpu.SemaphoreType.DMA,
            pltpu.SemaphoreType.DMA,
            pltpu.SemaphoreType.DMA,
            pltpu.SemaphoreType.DMA,
            pltpu.SemaphoreType.DMA,
        ],
    )
    def k(pl_hbm, f_hbm, t_hbm, out_hbm, tab_hbm, fi_v, ti_v,
          fs0, fs1, ts0, ts1, fr_v0, fr_v1, tr_v0, tr_v1,
          pin0, pin1, po0, po1, acc_v,
          sem_i, sem_g0, sem_g1, sem_p0, sem_p1, sem_o0, sem_o1):
        cid = lax.axis_index("c")
        sid = lax.axis_index("s")
        wid = sid * NC + cid
        base = wid * SLAB
        gsems = (sem_g0, sem_g1)
        psems = (sem_p0, sem_p1)
        osems = (sem_o0, sem_o1)
        fss = (fs0, fs1)
        tss = (ts0, ts1)
        frs = (fr_v0, fr_v1)
        trs = (tr_v0, tr_v1)
        pins = (pin0, pin1)
        pos = (po0, po1)
        acc_v[...] = jnp.zeros((L,), jnp.float32)
        rows16 = lax.iota(jnp.int32, 16)
        mytab = tab_hbm.at[cid]

        pltpu.async_copy(f_hbm.at[pl.ds(base, SLAB)], fi_v, sem_i)
        pltpu.async_copy(t_hbm.at[pl.ds(base, SLAB)], ti_v, sem_i)

        vbase = sid * VPS

        def p1_issue_in(p, slot):
            v0 = vbase + p * VB
            for comp in range(3):
                pltpu.async_copy(
                    pl_hbm.at[pl.ds(comp * PV + v0, VB)],
                    pins[slot].at[comp], psems[slot])

        def p1_drain_in(slot):
            for comp in range(3):
                pltpu.make_async_copy(
                    pl_hbm.at[pl.ds(0, VB)], pins[slot].at[comp],
                    psems[slot]).wait()

        def p1_drain_out(slot):
            pltpu.make_async_copy(
                pos[slot], mytab.at[pl.ds(0, VB // 2)], osems[slot]).wait()

        rp = jnp.right_shift(rows16, 1)
        cp = jnp.bitwise_and(rows16, 1)

        p1_issue_in(0, 0)
        p1_issue_in(1, 1)

        @pl.loop(0, PCH, step=NBUF)
        def _(p):
            for b in range(NBUF):
                step = p + b
                p1_drain_in(b)

                @pl.when(step >= NBUF)
                def _():
                    p1_drain_out(b)

                pin = pins[b]
                po = pos[b]

                @pl.loop(0, VB, step=L)
                def _(g):
                    xv = pin[0, pl.ds(g, L)]
                    yv = pin[1, pl.ds(g, L)]
                    zv = pin[2, pl.ds(g, L)]
                    rr = rp + jnp.right_shift(g, 1)
                    plsc.store_scatter(po, [rr, cp], xv)
                    plsc.store_scatter(po, [rr, cp + 2], yv)
                    plsc.store_scatter(po, [rr, cp + 4], zv)
                row0 = (vbase + step * VB) // 2
                pltpu.async_copy(
                    po, mytab.at[pl.ds(row0, VB // 2)], osems[b])

                @pl.when(step + NBUF < PCH)
                def _():
                    p1_issue_in(step + NBUF, b)

        p1_drain_out(0)
        p1_drain_out(1)
        plsc.subcore_barrier()

        pltpu.make_async_copy(f_hbm.at[pl.ds(base, SLAB)], fi_v, sem_i).wait()
        pltpu.make_async_copy(t_hbm.at[pl.ds(base, SLAB)], ti_v, sem_i).wait()

        def stage_issue(chunk, slot):
            off = chunk * CHUNK

            @pl.loop(0, CHUNK, step=L)
            def _(g):
                fss[slot][pl.ds(g, L)] = jnp.right_shift(
                    fi_v[pl.ds(off + g, L)], 1)
                tss[slot][pl.ds(g, L)] = jnp.right_shift(
                    ti_v[pl.ds(off + g, L)], 1)
            pltpu.async_copy(mytab.at[fss[slot]], frs[slot], gsems[slot])
            pltpu.async_copy(mytab.at[tss[slot]], trs[slot], gsems[slot])

        def drain(slot):
            pltpu.make_async_copy(
                mytab.at[fss[slot]], frs[slot], gsems[slot]).wait()
            pltpu.make_async_copy(
                mytab.at[tss[slot]], trs[slot], gsems[slot]).wait()

        stage_issue(0, 0)
        stage_issue(1, 1)

        @pl.loop(0, CPW, step=NBUF)
        def _(j):
            for b in range(NBUF):
                chunk = j + b
                drain(b)
                fr = frs[b]
                tr = trs[b]
                off = chunk * CHUNK

                @pl.loop(0, CHUNK, step=L)
                def _(r):
                    ridx = rows16 + r
                    mf = jnp.bitwise_and(fi_v[pl.ds(off + r, L)], 1)
                    mt = jnp.bitwise_and(ti_v[pl.ds(off + r, L)], 1)
                    xf = plsc.load_gather(fr, [ridx, mf])
                    yf = plsc.load_gather(fr, [ridx, mf + 2])
                    zf = plsc.load_gather(fr, [ridx, mf + 4])
                    xt = plsc.load_gather(tr, [ridx, mt])
                    yt = plsc.load_gather(tr, [ridx, mt + 2])
                    zt = plsc.load_gather(tr, [ridx, mt + 4])
                    sx = xt + xf
                    sy = yt - yf
                    sz = zt - zf
                    ss = sx * sx + sy * sy + sz * sz
                    ss = jnp.maximum(ss, 1e-30)
                    bits = plsc.bitcast(ss, jnp.int32)
                    bits = 0x5F3759DF - jnp.right_shift(bits, 1)
                    y = plsc.bitcast(bits, jnp.float32)
                    for _ in range(2):
                        y = y * (1.5 - 0.5 * ss * y * y)
                    acc_v[...] = acc_v[...] + ss * y

                @pl.when(chunk + NBUF < CPW)
                def _():
                    stage_issue(chunk + NBUF, b)

        pltpu.async_copy(acc_v, out_hbm.at[wid], sem_i).wait()

    return k(planes, fidx, tidx)


def kernel(vt, mapping_table):
    planes = jnp.pad(vt.T, ((0, 0), (0, PV - N_VERTS))).reshape(3 * PV)
    pad = jnp.full((M_PAD - N_MAPS,), N_VERTS, jnp.int32)
    fidx = jnp.concatenate([mapping_table[:, 0], pad])
    tidx = jnp.concatenate([mapping_table[:, 1], pad])
    parts, _ = _sc_loss_parts(planes, fidx, tidx)
    return jnp.sum(parts) / N_MAPS
